# baseline (speedup 1.0000x reference)
import functools

import jax
import jax.numpy as jnp
from jax import lax
from jax.experimental import pallas as pl
from jax.experimental.pallas import tpu as pltpu
from jax.experimental.pallas import tpu_sc as plsc

N = 819200
D = 64
VOCAB = 1000000

BP = 8192
VBLK = (VOCAB + 2 * BP - 1) // (2 * BP)
VPAD = VBLK * BP

_info = plsc.get_sparse_core_info()
NC, NS, L = _info.num_cores, _info.num_subcores, _info.num_lanes
NW = NC * NS

CHUNK = 128
ROWS_PER_W = N // NW
CHUNKS_PER_W = ROWS_PER_W // CHUNK
G = 8
GROUP = G * CHUNK
STEPS = CHUNKS_PER_W // G


def _relayout_body(x0_ref, x1_ref, eye_ref, o_ref):
    xcat = jnp.concatenate([x0_ref[...], x1_ref[...]], axis=0)
    o_ref[...] = lax.dot_general(
        xcat, eye_ref[...], (((0,), (0,)), ((), ())),
        preferred_element_type=jnp.float32,
    )


def _tc_relayout(tableT, eye2):
    return pl.pallas_call(
        _relayout_body,
        out_shape=jax.ShapeDtypeStruct((VPAD, 2 * D), jnp.float32),
        grid=(VBLK,),
        in_specs=[
            pl.BlockSpec((D, BP), lambda j: (0, 2 * j)),
            pl.BlockSpec(
                (D, BP),
                lambda j: (0, jnp.minimum(2 * j + 1, VOCAB // BP - 1)),
            ),
            pl.BlockSpec((2 * D, 2 * D), lambda j: (0, 0)),
        ],
        out_specs=pl.BlockSpec((BP, 2 * D), lambda j: (j, 0)),
    )(tableT, tableT, eye2)


K = 5
NCH = N // K
ROWS_PER_WC = NCH // NW
CHUNKS_PER_WC = ROWS_PER_WC // CHUNK
STEPS_C = CHUNKS_PER_WC // G


def _sc_gather_chunk(table_lin, idx3):
    mesh = plsc.VectorSubcoreMesh(core_axis_name="c", subcore_axis_name="s")

    @functools.partial(
        pl.kernel,
        mesh=mesh,
        out_type=jax.ShapeDtypeStruct((NCH // 2, 2 * D), jnp.float32),
        compiler_params=pltpu.CompilerParams(use_tc_tiling_on_sc=False),
        scratch_types=[
            pltpu.VMEM((CHUNKS_PER_WC, CHUNK), jnp.int32),
            pltpu.VMEM((GROUP, D), jnp.float32),
            pltpu.SemaphoreType.DMA,
        ],
    )
    def gather_kernel(table_hbm, idx_hbm, out_hbm, idx_v, rows_v, sem):
        wid = lax.axis_index("s") * NC + lax.axis_index("c")
        pltpu.sync_copy(idx_hbm.at[wid], idx_v)

        def body(i, carry):
            base_chunk = i * G
            copies = [
                pltpu.async_copy(
                    table_hbm.at[idx_v.at[base_chunk + j]],
                    rows_v.at[pl.ds(j * CHUNK, CHUNK)],
                    sem,
                )
                for j in range(G)
            ]
            for c in copies:
                c.wait()
            c0 = wid * ROWS_PER_WC + i * GROUP
            blk = c0 // (2 * BP)
            rem = c0 % (2 * BP)
            pltpu.sync_copy(
                rows_v,
                out_hbm.at[pl.ds(blk * BP + rem % BP, GROUP),
                           pl.ds((rem // BP) * D, D)],
            )
            return carry

        lax.fori_loop(0, STEPS_C, body, 0)

    return gather_kernel(table_lin, idx3)


def _mm_body(x_ref, wblk_ref, b_ref, o_ref):
    zz = lax.dot_general(
        wblk_ref[...], x_ref[...], (((1,), (1,)), ((), ())),
        preferred_element_type=jnp.float32,
    )
    o_ref[:, 0:BP] = zz[0:D, :] + b_ref[...]
    o_ref[:, BP:2 * BP] = zz[D:2 * D, :] + b_ref[...]


def _mm_body_acc(x_ref, wblk_ref, b_ref, prev_ref, o_ref):
    del prev_ref
    _mm_body(x_ref, wblk_ref, b_ref, o_ref)


NB_C = NCH // (2 * BP)


def _tc_matmul_chunk(k, x2c, Wblk, b2, prev):
    out_spec = pl.BlockSpec((D, 2 * BP), lambda j, k=k: (0, k * NB_C + j))
    x_spec = pl.BlockSpec((BP, 2 * D), lambda j: (j, 0))
    w_spec = pl.BlockSpec((2 * D, 2 * D), lambda j: (0, 0))
    b_spec = pl.BlockSpec((D, 1), lambda j: (0, 0))
    if prev is None:
        return pl.pallas_call(
            _mm_body,
            out_shape=jax.ShapeDtypeStruct((D, N), jnp.float32),
            grid=(NB_C,),
            in_specs=[x_spec, w_spec, b_spec],
            out_specs=out_spec,
        )(x2c, Wblk, b2)
    return pl.pallas_call(
        _mm_body_acc,
        out_shape=jax.ShapeDtypeStruct((D, N), jnp.float32),
        grid=(NB_C,),
        in_specs=[x_spec, w_spec, b_spec,
                  pl.BlockSpec((8, 128), lambda j: (0, 0))],
        out_specs=out_spec,
        input_output_aliases={3: 0},
    )(x2c, Wblk, b2, prev)


def kernel(indices, offsets, table, W, b):
    del offsets
    blk = indices // (2 * BP)
    rem = indices % (2 * BP)
    idx2 = (blk * BP + (rem % BP)) * 2 + rem // BP
    idx4 = idx2.reshape(K, NW, CHUNKS_PER_WC, CHUNK)
    eye2 = jnp.eye(2 * D, dtype=jnp.float32)
    wblk = jnp.kron(jnp.eye(2, dtype=jnp.float32), W)
    b2 = b.reshape(D, 1)
    table2 = _tc_relayout(table.T, eye2)
    table_lin = table2.reshape(2 * VPAD, D)
    out_t = None
    for k in range(K):
        x2c = _sc_gather_chunk(table_lin, idx4[k])
        out_t = _tc_matmul_chunk(k, x2c, wblk, b2, out_t)
    return out_t.T


# speedup vs baseline: 1.0324x; 1.0324x over previous
import functools

import jax
import jax.numpy as jnp
from jax import lax
from jax.experimental import pallas as pl
from jax.experimental.pallas import tpu as pltpu
from jax.experimental.pallas import tpu_sc as plsc

N = 819200
D = 64
VOCAB = 1000000

BP = 8192
VBLK = (VOCAB + 2 * BP - 1) // (2 * BP)
VPAD = VBLK * BP

_info = plsc.get_sparse_core_info()
NC, NS, L = _info.num_cores, _info.num_subcores, _info.num_lanes
NW = NC * NS

CHUNK = 128
ROWS_PER_W = N // NW
CHUNKS_PER_W = ROWS_PER_W // CHUNK
G = 8
GROUP = G * CHUNK
STEPS = CHUNKS_PER_W // G


def _relayout_body(x0_ref, x1_ref, eye_ref, o_ref):
    xcat = jnp.concatenate([x0_ref[...], x1_ref[...]], axis=0)
    o_ref[...] = lax.dot_general(
        xcat, eye_ref[...], (((0,), (0,)), ((), ())),
        preferred_element_type=jnp.float32,
    )


def _tc_relayout(tableT, eye2):
    return pl.pallas_call(
        _relayout_body,
        out_shape=jax.ShapeDtypeStruct((VPAD, 2 * D), jnp.float32),
        grid=(VBLK,),
        in_specs=[
            pl.BlockSpec((D, BP), lambda j: (0, 2 * j)),
            pl.BlockSpec(
                (D, BP),
                lambda j: (0, jnp.minimum(2 * j + 1, VOCAB // BP - 1)),
            ),
            pl.BlockSpec((2 * D, 2 * D), lambda j: (0, 0)),
        ],
        out_specs=pl.BlockSpec((BP, 2 * D), lambda j: (j, 0)),
    )(tableT, tableT, eye2)


def _sc_gather(table_lin, idx3):
    mesh = plsc.VectorSubcoreMesh(core_axis_name="c", subcore_axis_name="s")

    @functools.partial(
        pl.kernel,
        mesh=mesh,
        out_type=jax.ShapeDtypeStruct((N // 2, 2 * D), jnp.float32),
        compiler_params=pltpu.CompilerParams(use_tc_tiling_on_sc=False),
        scratch_types=[
            pltpu.VMEM((CHUNKS_PER_W, CHUNK), jnp.int32),
            pltpu.VMEM((GROUP, D), jnp.float32),
            pltpu.SemaphoreType.DMA,
        ],
    )
    def gather_kernel(table_hbm, idx_hbm, out_hbm, idx_v, rows_v, sem):
        wid = lax.axis_index("s") * NC + lax.axis_index("c")
        pltpu.sync_copy(idx_hbm.at[wid], idx_v)

        def body(i, carry):
            base_chunk = i * G
            copies = [
                pltpu.async_copy(
                    table_hbm.at[idx_v.at[base_chunk + j]],
                    rows_v.at[pl.ds(j * CHUNK, CHUNK)],
                    sem,
                )
                for j in range(G)
            ]
            for c in copies:
                c.wait()
            c0 = wid * ROWS_PER_W + i * GROUP
            blk = c0 // (2 * BP)
            rem = c0 % (2 * BP)
            pltpu.sync_copy(
                rows_v,
                out_hbm.at[pl.ds(blk * BP + rem % BP, GROUP),
                           pl.ds((rem // BP) * D, D)],
            )
            return carry

        lax.fori_loop(0, STEPS, body, 0)

    return gather_kernel(table_lin, idx3)


def _mm_body(x_ref, wblk_ref, b_ref, o_ref):
    zz = lax.dot_general(
        wblk_ref[...], x_ref[...], (((1,), (1,)), ((), ())),
        preferred_element_type=jnp.float32,
    )
    o_ref[:, 0:BP] = zz[0:D, :] + b_ref[...]
    o_ref[:, BP:2 * BP] = zz[D:2 * D, :] + b_ref[...]


def _tc_matmul(x2, Wblk, b2):
    return pl.pallas_call(
        _mm_body,
        out_shape=jax.ShapeDtypeStruct((D, N), jnp.float32),
        grid=(N // (2 * BP),),
        in_specs=[
            pl.BlockSpec((BP, 2 * D), lambda j: (j, 0)),
            pl.BlockSpec((2 * D, 2 * D), lambda j: (0, 0)),
            pl.BlockSpec((D, 1), lambda j: (0, 0)),
        ],
        out_specs=pl.BlockSpec((D, 2 * BP), lambda j: (0, j)),
    )(x2, Wblk, b2)


def kernel(indices, offsets, table, W, b):
    del offsets
    blk = indices // (2 * BP)
    rem = indices % (2 * BP)
    idx2 = (blk * BP + (rem % BP)) * 2 + rem // BP
    idx3 = idx2.reshape(NW, CHUNKS_PER_W, CHUNK)
    eye2 = jnp.eye(2 * D, dtype=jnp.float32)
    wblk = jnp.kron(jnp.eye(2, dtype=jnp.float32), W)
    b2 = b.reshape(D, 1)
    table2 = _tc_relayout(table.T, eye2)
    table_lin = table2.reshape(2 * VPAD, D)
    x2 = _sc_gather(table_lin, idx3)
    out_t = _tc_matmul(x2, wblk, b2)
    return out_t.T
